# baseline (speedup 1.0000x reference)
import functools
import math

import jax
import jax.numpy as jnp
from jax import lax
from jax.experimental import pallas as pl
from jax.experimental.pallas import tpu as pltpu
from jax.experimental.pallas import tpu_sc as plsc

D_MODEL = 64
SCALE = math.sqrt(D_MODEL)
NUM_WORKERS = 32
CHUNK = 512
NBUF = 3


@functools.partial(jax.jit, static_argnames=("batch",))
def _embed_lookup(x_flat, table, batch):
    b_per_w = batch // NUM_WORKERS
    n_chunks = b_per_w // CHUNK
    mesh = plsc.VectorSubcoreMesh(core_axis_name="c", subcore_axis_name="s")

    @functools.partial(
        pl.kernel,
        mesh=mesh,
        out_type=jax.ShapeDtypeStruct((batch, D_MODEL), jnp.float32),
        scratch_types=[
            pltpu.VMEM((b_per_w,), jnp.int32),
            pltpu.VMEM((NBUF, CHUNK, D_MODEL), jnp.float32),
            pltpu.SemaphoreType.DMA((NBUF,)),
            pltpu.SemaphoreType.DMA((NBUF,)),
        ],
        compiler_params=pltpu.CompilerParams(use_tc_tiling_on_sc=False),
    )
    def k(x_hbm, t_hbm, out_hbm, idx_all, rows, gsem, osem):
        cid = lax.axis_index("c")
        sid = lax.axis_index("s")
        wid = sid * 2 + cid
        base = wid * b_per_w

        pltpu.sync_copy(x_hbm.at[pl.ds(base, b_per_w)], idx_all)

        def gather_start(chunk):
            buf = lax.rem(chunk, NBUF)
            pltpu.async_copy(
                t_hbm.at[idx_all.at[pl.ds(chunk * CHUNK, CHUNK)]],
                rows.at[buf],
                gsem.at[buf],
            )

        def gather_wait(chunk):
            buf = lax.rem(chunk, NBUF)
            pltpu.make_async_copy(
                t_hbm.at[idx_all.at[pl.ds(chunk * CHUNK, CHUNK)]],
                rows.at[buf],
                gsem.at[buf],
            ).wait()

        def out_start(chunk, buf):
            pltpu.async_copy(
                rows.at[buf],
                out_hbm.at[pl.ds(base + chunk * CHUNK, CHUNK)],
                osem.at[buf],
            )

        def out_wait(buf):
            pltpu.make_async_copy(
                rows.at[buf], out_hbm.at[pl.ds(0, CHUNK)], osem.at[buf]
            ).wait()

        gather_start(0)
        gather_start(1)

        def body(i, carry):
            buf = lax.rem(i, NBUF)
            gather_wait(i)

            def scale_row(r, c):
                for j in range(D_MODEL // 16):
                    sl = pl.ds(j * 16, 16)
                    rows[buf, r, sl] = rows[buf, r, sl] * SCALE
                return c

            lax.fori_loop(0, CHUNK, scale_row, 0)

            out_start(i, buf)

            @pl.when(i >= 1)
            def _():
                out_wait(lax.rem(i + 2, NBUF))

            @pl.when(i + 2 < n_chunks)
            def _():
                gather_start(i + 2)

            return carry

        lax.fori_loop(0, n_chunks, body, 0)
        out_wait(lax.rem(n_chunks - 1, NBUF))

    return k(x_flat, table)


def kernel(x, table):
    batch = x.size
    out = _embed_lookup(x.reshape(-1).astype(jnp.int32), table, batch)
    return out.reshape(x.shape + (D_MODEL,))


# speedup vs baseline: 1.3269x; 1.3269x over previous
import functools
import math

import jax
import jax.numpy as jnp
from jax import lax
from jax.experimental import pallas as pl
from jax.experimental.pallas import tpu as pltpu
from jax.experimental.pallas import tpu_sc as plsc

D_MODEL = 64
SCALE = math.sqrt(D_MODEL)
NUM_WORKERS = 32
CHUNK = 320
NBUF = 4
PREF = 2


@functools.partial(jax.jit, static_argnames=("batch",))
def _embed_lookup(x_flat, table, batch):
    b_per_w = batch // NUM_WORKERS
    n_chunks = b_per_w // CHUNK
    n_rounds = n_chunks // NBUF
    mesh = plsc.VectorSubcoreMesh(core_axis_name="c", subcore_axis_name="s")

    @functools.partial(
        pl.kernel,
        mesh=mesh,
        out_type=jax.ShapeDtypeStruct((batch, D_MODEL), jnp.float32),
        scratch_types=[
            pltpu.VMEM((b_per_w,), jnp.int32),
            pltpu.VMEM((NBUF, CHUNK, D_MODEL), jnp.float32),
            pltpu.SemaphoreType.DMA((NBUF,)),
            pltpu.SemaphoreType.DMA((NBUF,)),
        ],
        compiler_params=pltpu.CompilerParams(use_tc_tiling_on_sc=False),
    )
    def k(x_hbm, t_hbm, out_hbm, idx_all, rows, gsem, osem):
        cid = lax.axis_index("c")
        sid = lax.axis_index("s")
        wid = sid * 2 + cid
        base = wid * b_per_w

        pltpu.sync_copy(x_hbm.at[pl.ds(base, b_per_w)], idx_all)

        def gather_start(chunk, buf):
            pltpu.async_copy(
                t_hbm.at[idx_all.at[pl.ds(chunk * CHUNK, CHUNK)]],
                rows.at[buf],
                gsem.at[buf],
            )

        def gather_wait(chunk, buf):
            pltpu.make_async_copy(
                t_hbm.at[idx_all.at[pl.ds(chunk * CHUNK, CHUNK)]],
                rows.at[buf],
                gsem.at[buf],
            ).wait()

        def out_start(chunk, buf):
            pltpu.async_copy(
                rows.at[buf],
                out_hbm.at[pl.ds(base + chunk * CHUNK, CHUNK)],
                osem.at[buf],
            )

        def out_wait(buf):
            pltpu.make_async_copy(
                rows.at[buf], out_hbm.at[pl.ds(0, CHUNK)], osem.at[buf]
            ).wait()

        def scale_buf(buf):
            def scale_rows(r, c):
                for u in range(2):
                    for j in range(D_MODEL // 16):
                        sl = pl.ds(j * 16, 16)
                        rows[buf, r * 2 + u, sl] = rows[buf, r * 2 + u, sl] * SCALE
                return c

            lax.fori_loop(0, CHUNK // 2, scale_rows, 0)

        gather_start(0, 0)
        gather_start(1, 1)

        def round_body(rnd, carry):
            for b in range(NBUF):
                chunk = rnd * NBUF + b
                gather_wait(chunk, b)
                scale_buf(b)
                out_start(chunk, b)
                gb = (b + PREF) % NBUF

                @pl.when(chunk + PREF >= NBUF)
                def _():
                    out_wait(gb)

                @pl.when(chunk + PREF < n_chunks)
                def _():
                    gather_start(chunk + PREF, gb)

            return carry

        lax.fori_loop(0, n_rounds, round_body, 0)
        out_wait((n_chunks - 2) % NBUF)
        out_wait((n_chunks - 1) % NBUF)

    return k(x_flat, table)


def kernel(x, table):
    batch = x.size
    out = _embed_lookup(x.reshape(-1).astype(jnp.int32), table, batch)
    return out.reshape(x.shape + (D_MODEL,))
